# baseline (speedup 1.0000x reference)
import jax
import jax.numpy as jnp
from jax import lax
from jax.experimental import pallas as pl
from jax.experimental.pallas import tpu as pltpu
from jax.experimental.pallas import tpu_sc as plsc

B = 16384
D = 128
NC = 2
NS = 16
NW = NC * NS
CH = 128
SLICES = (12288, 4096)


def _fuse_math(semb, aemb, wa_ref, ba_ref, fw_ref, bf_ref):
    w1 = fw_ref[:D, :]
    w2 = fw_ref[D:, :]
    m = jnp.dot(wa_ref[...], w2, preferred_element_type=jnp.float32)
    c = jnp.dot(ba_ref[...], w2, preferred_element_type=jnp.float32) + bf_ref[...]
    return (
        jnp.dot(semb, w1, preferred_element_type=jnp.float32)
        + jnp.dot(aemb, m, preferred_element_type=jnp.float32)
        + c
    )


def _fuse_body(semb_ref, aemb_ref, wa_ref, ba_ref, fw_ref, bf_ref, out_ref):
    out_ref[...] = _fuse_math(semb_ref[...], aemb_ref[...], wa_ref, ba_ref,
                              fw_ref, bf_ref)


def _fuse_body_aliased(prev_ref, semb_ref, aemb_ref, wa_ref, ba_ref, fw_ref,
                       bf_ref, out_ref):
    del prev_ref
    out_ref[...] = _fuse_math(semb_ref[...], aemb_ref[...], wa_ref, ba_ref,
                              fw_ref, bf_ref)


def _make_sc_gather(row0, nch):
    bpw = nch * CH

    def _sc_gather(node_hbm, attr_hbm, stab_hbm, atab_hbm, semb_hbm, aemb_hbm,
                   nidx_v, aidx_v, srow0, srow1, arow0, arow1, atab_sh,
                   sem_s0, sem_s1, sem_a0, sem_a1):
        wid = lax.axis_index("s") * NC + lax.axis_index("c")
        sid = lax.axis_index("s")
        base = wid * bpw
        srow = (srow0, srow1)
        arow = (arow0, arow1)
        sem_s = (sem_s0, sem_s1)
        sem_a = (sem_a0, sem_a1)
        @pl.when(sid == 0)
        def _stage():
            pltpu.sync_copy(atab_hbm, atab_sh)

        r = row0 + wid * nch
        for j in range(nch):
            pltpu.sync_copy(node_hbm.at[r + j], nidx_v.at[j])
            pltpu.sync_copy(attr_hbm.at[r + j], aidx_v.at[j])
        plsc.subcore_barrier()
        copies = [None, None]
        for j in range(nch + 1):
            if j < nch:
                s = j % 2
                cs = pltpu.async_copy(stab_hbm.at[nidx_v.at[j]], srow[s], sem_s[s])
                ca = pltpu.async_copy(atab_sh.at[aidx_v.at[j]], arow[s], sem_a[s])
                copies[s] = (cs, ca)
            if j >= 1:
                p = (j - 1) % 2
                cs, ca = copies[p]
                cs.wait()
                ca.wait()
                off = base + (j - 1) * CH
                pltpu.sync_copy(srow[p], semb_hbm.at[pl.ds(off, CH)])
                pltpu.sync_copy(arow[p], aemb_hbm.at[pl.ds(off, CH)])

    return _sc_gather


def kernel(node_ids, attrs, struct_table, attr_table, attr_fc_w, attr_fc_b, fusion_w, fusion_b):
    mesh = plsc.VectorSubcoreMesh(core_axis_name="c", subcore_axis_name="s",
                                  num_cores=NC, num_subcores=NS)
    node2d = node_ids.reshape(B // CH, CH)
    attr2d = attrs.reshape(B // CH, CH)

    def scratch(nch):
        return [
            pltpu.VMEM((nch, CH), jnp.int32),
            pltpu.VMEM((nch, CH), jnp.int32),
            pltpu.VMEM((CH, D), jnp.float32),
            pltpu.VMEM((CH, D), jnp.float32),
            pltpu.VMEM((CH, D), jnp.float32),
            pltpu.VMEM((CH, D), jnp.float32),
            pltpu.VMEM_SHARED((1001, D), jnp.float32),
            pltpu.SemaphoreType.DMA,
            pltpu.SemaphoreType.DMA,
            pltpu.SemaphoreType.DMA,
            pltpu.SemaphoreType.DMA,
        ]

    slices = []
    off = 0
    for bh in SLICES:
        nch = bh // NW // CH
        slices.append((off, bh,
                       pl.kernel(
                           _make_sc_gather(off // CH, nch),
                           out_type=[
                               jax.ShapeDtypeStruct((bh, D), jnp.float32),
                               jax.ShapeDtypeStruct((bh, D), jnp.float32),
                           ],
                           mesh=mesh,
                           scratch_types=scratch(nch),
                       )(node2d, attr2d, struct_table, attr_table)))
        off += bh

    ba2 = attr_fc_b.reshape(1, D)
    bf2 = fusion_b.reshape(1, D)
    out = None
    for k, (off, bh, (semb_h, aemb_h)) in enumerate(slices):
        BLK = min(4096, bh // 2)
        nblk = bh // BLK
        blk0 = off // BLK
        if k == 0:
            out = pl.pallas_call(
                _fuse_body,
                grid=(nblk,),
                in_specs=[
                    pl.BlockSpec((BLK, D), lambda i: (i, 0)),
                    pl.BlockSpec((BLK, D), lambda i: (i, 0)),
                    pl.BlockSpec((D, D), lambda i: (0, 0)),
                    pl.BlockSpec((1, D), lambda i: (0, 0)),
                    pl.BlockSpec((2 * D, D), lambda i: (0, 0)),
                    pl.BlockSpec((1, D), lambda i: (0, 0)),
                ],
                out_specs=pl.BlockSpec((BLK, D), lambda i: (i, 0)),
                out_shape=jax.ShapeDtypeStruct((B, D), jnp.float32),
            )(semb_h, aemb_h, attr_fc_w, ba2, fusion_w, bf2)
        else:
            out = pl.pallas_call(
                _fuse_body_aliased,
                grid=(nblk,),
                in_specs=[
                    pl.BlockSpec(memory_space=pltpu.MemorySpace.HBM),
                    pl.BlockSpec((BLK, D), lambda i: (i, 0)),
                    pl.BlockSpec((BLK, D), lambda i: (i, 0)),
                    pl.BlockSpec((D, D), lambda i: (0, 0)),
                    pl.BlockSpec((1, D), lambda i: (0, 0)),
                    pl.BlockSpec((2 * D, D), lambda i: (0, 0)),
                    pl.BlockSpec((1, D), lambda i: (0, 0)),
                ],
                out_specs=pl.BlockSpec((BLK, D),
                                       lambda i, blk0=blk0: (i + blk0, 0)),
                out_shape=jax.ShapeDtypeStruct((B, D), jnp.float32),
                input_output_aliases={0: 0},
            )(out, semb_h, aemb_h, attr_fc_w, ba2, fusion_w, bf2)
    return out


# speedup vs baseline: 1.0250x; 1.0250x over previous
import jax
import jax.numpy as jnp
from jax import lax
from jax.experimental import pallas as pl
from jax.experimental.pallas import tpu as pltpu
from jax.experimental.pallas import tpu_sc as plsc

B = 16384
D = 128
NC = 2
NS = 16
NW = NC * NS
CH = 128
SLICES = (8192, 8192)


def _fuse_math(semb, aemb, wa_ref, ba_ref, fw_ref, bf_ref):
    w1 = fw_ref[:D, :]
    w2 = fw_ref[D:, :]
    m = jnp.dot(wa_ref[...], w2, preferred_element_type=jnp.float32)
    c = jnp.dot(ba_ref[...], w2, preferred_element_type=jnp.float32) + bf_ref[...]
    return (
        jnp.dot(semb, w1, preferred_element_type=jnp.float32)
        + jnp.dot(aemb, m, preferred_element_type=jnp.float32)
        + c
    )


def _fuse_body(semb_ref, aemb_ref, wa_ref, ba_ref, fw_ref, bf_ref, out_ref):
    out_ref[...] = _fuse_math(semb_ref[...], aemb_ref[...], wa_ref, ba_ref,
                              fw_ref, bf_ref)


def _fuse_body_aliased(prev_ref, semb_ref, aemb_ref, wa_ref, ba_ref, fw_ref,
                       bf_ref, out_ref):
    del prev_ref
    out_ref[...] = _fuse_math(semb_ref[...], aemb_ref[...], wa_ref, ba_ref,
                              fw_ref, bf_ref)


def _make_sc_gather(row0, nch):
    bpw = nch * CH

    def _sc_gather(node_hbm, attr_hbm, stab_hbm, atab_hbm, semb_hbm, aemb_hbm,
                   nidx_v, aidx_v, srow0, srow1, arow0, arow1, atab_sh,
                   sem_s0, sem_s1, sem_a0, sem_a1):
        wid = lax.axis_index("s") * NC + lax.axis_index("c")
        sid = lax.axis_index("s")
        base = wid * bpw
        srow = (srow0, srow1)
        arow = (arow0, arow1)
        sem_s = (sem_s0, sem_s1)
        sem_a = (sem_a0, sem_a1)
        @pl.when(sid == 0)
        def _stage():
            pltpu.sync_copy(atab_hbm, atab_sh)

        r = row0 + wid * nch
        for j in range(nch):
            pltpu.sync_copy(node_hbm.at[r + j], nidx_v.at[j])
            pltpu.sync_copy(attr_hbm.at[r + j], aidx_v.at[j])
        plsc.subcore_barrier()
        copies = [None, None]
        for j in range(nch + 1):
            if j < nch:
                s = j % 2
                cs = pltpu.async_copy(stab_hbm.at[nidx_v.at[j]], srow[s], sem_s[s])
                ca = pltpu.async_copy(atab_sh.at[aidx_v.at[j]], arow[s], sem_a[s])
                copies[s] = (cs, ca)
            if j >= 1:
                p = (j - 1) % 2
                cs, ca = copies[p]
                cs.wait()
                ca.wait()
                off = base + (j - 1) * CH
                pltpu.sync_copy(srow[p], semb_hbm.at[pl.ds(off, CH)])
                pltpu.sync_copy(arow[p], aemb_hbm.at[pl.ds(off, CH)])

    return _sc_gather


def kernel(node_ids, attrs, struct_table, attr_table, attr_fc_w, attr_fc_b, fusion_w, fusion_b):
    mesh = plsc.VectorSubcoreMesh(core_axis_name="c", subcore_axis_name="s",
                                  num_cores=NC, num_subcores=NS)
    node2d = node_ids.reshape(B // CH, CH)
    attr2d = attrs.reshape(B // CH, CH)

    def scratch(nch):
        return [
            pltpu.VMEM((nch, CH), jnp.int32),
            pltpu.VMEM((nch, CH), jnp.int32),
            pltpu.VMEM((CH, D), jnp.float32),
            pltpu.VMEM((CH, D), jnp.float32),
            pltpu.VMEM((CH, D), jnp.float32),
            pltpu.VMEM((CH, D), jnp.float32),
            pltpu.VMEM_SHARED((1001, D), jnp.float32),
            pltpu.SemaphoreType.DMA,
            pltpu.SemaphoreType.DMA,
            pltpu.SemaphoreType.DMA,
            pltpu.SemaphoreType.DMA,
        ]

    slices = []
    off = 0
    for bh in SLICES:
        nch = bh // NW // CH
        slices.append((off, bh,
                       pl.kernel(
                           _make_sc_gather(off // CH, nch),
                           out_type=[
                               jax.ShapeDtypeStruct((bh, D), jnp.float32),
                               jax.ShapeDtypeStruct((bh, D), jnp.float32),
                           ],
                           mesh=mesh,
                           scratch_types=scratch(nch),
                       )(node2d, attr2d, struct_table, attr_table)))
        off += bh

    ba2 = attr_fc_b.reshape(1, D)
    bf2 = fusion_b.reshape(1, D)
    out = None
    for k, (off, bh, (semb_h, aemb_h)) in enumerate(slices):
        BLK = min(4096, bh // 2)
        nblk = bh // BLK
        blk0 = off // BLK
        if k == 0:
            out = pl.pallas_call(
                _fuse_body,
                grid=(nblk,),
                in_specs=[
                    pl.BlockSpec((BLK, D), lambda i: (i, 0)),
                    pl.BlockSpec((BLK, D), lambda i: (i, 0)),
                    pl.BlockSpec((D, D), lambda i: (0, 0)),
                    pl.BlockSpec((1, D), lambda i: (0, 0)),
                    pl.BlockSpec((2 * D, D), lambda i: (0, 0)),
                    pl.BlockSpec((1, D), lambda i: (0, 0)),
                ],
                out_specs=pl.BlockSpec((BLK, D), lambda i: (i, 0)),
                out_shape=jax.ShapeDtypeStruct((B, D), jnp.float32),
            )(semb_h, aemb_h, attr_fc_w, ba2, fusion_w, bf2)
        else:
            out = pl.pallas_call(
                _fuse_body_aliased,
                grid=(nblk,),
                in_specs=[
                    pl.BlockSpec(memory_space=pltpu.MemorySpace.HBM),
                    pl.BlockSpec((BLK, D), lambda i: (i, 0)),
                    pl.BlockSpec((BLK, D), lambda i: (i, 0)),
                    pl.BlockSpec((D, D), lambda i: (0, 0)),
                    pl.BlockSpec((1, D), lambda i: (0, 0)),
                    pl.BlockSpec((2 * D, D), lambda i: (0, 0)),
                    pl.BlockSpec((1, D), lambda i: (0, 0)),
                ],
                out_specs=pl.BlockSpec((BLK, D),
                                       lambda i, blk0=blk0: (i + blk0, 0)),
                out_shape=jax.ShapeDtypeStruct((B, D), jnp.float32),
                input_output_aliases={0: 0},
            )(out, semb_h, aemb_h, attr_fc_w, ba2, fusion_w, bf2)
    return out


# speedup vs baseline: 1.0647x; 1.0387x over previous
import jax
import jax.numpy as jnp
from jax import lax
from jax.experimental import pallas as pl
from jax.experimental.pallas import tpu as pltpu
from jax.experimental.pallas import tpu_sc as plsc

B = 16384
D = 128
NC = 2
NS = 16
NW = NC * NS
CH = 128
SLICES = (8192, 8192)


def _fuse_math(semb, aemb, wa_ref, ba_ref, fw_ref, bf_ref):
    w1 = fw_ref[:D, :]
    w2 = fw_ref[D:, :]
    m = jnp.dot(wa_ref[...], w2, preferred_element_type=jnp.float32)
    c = jnp.dot(ba_ref[...], w2, preferred_element_type=jnp.float32) + bf_ref[...]
    return (
        jnp.dot(semb, w1, preferred_element_type=jnp.float32)
        + jnp.dot(aemb, m, preferred_element_type=jnp.float32)
        + c
    )


def _fuse_body(semb_ref, aemb_ref, wa_ref, ba_ref, fw_ref, bf_ref, out_ref):
    out_ref[...] = _fuse_math(semb_ref[...], aemb_ref[...], wa_ref, ba_ref,
                              fw_ref, bf_ref)


def _fuse_body_aliased(prev_ref, semb_ref, aemb_ref, wa_ref, ba_ref, fw_ref,
                       bf_ref, out_ref):
    del prev_ref
    out_ref[...] = _fuse_math(semb_ref[...], aemb_ref[...], wa_ref, ba_ref,
                              fw_ref, bf_ref)


def _make_sc_gather(row0, nch):
    bpw = nch * CH

    def _sc_gather(node_hbm, attr_hbm, stab_hbm, atab_hbm, semb_hbm, aemb_hbm,
                   nidx_v, aidx_v, srow0, srow1, arow0, arow1, atab_sh,
                   sem_s0, sem_s1, sem_a0, sem_a1):
        wid = lax.axis_index("s") * NC + lax.axis_index("c")
        sid = lax.axis_index("s")
        base = wid * bpw
        srow = (srow0, srow1)
        arow = (arow0, arow1)
        sem_s = (sem_s0, sem_s1)
        sem_a = (sem_a0, sem_a1)
        @pl.when(sid == 0)
        def _stage():
            pltpu.sync_copy(atab_hbm, atab_sh)

        r = row0 + wid * nch
        pltpu.sync_copy(node_hbm.at[pl.ds(r, nch)], nidx_v)
        pltpu.sync_copy(attr_hbm.at[pl.ds(r, nch)], aidx_v)
        plsc.subcore_barrier()
        copies = [None, None]
        for j in range(nch + 1):
            if j < nch:
                s = j % 2
                cs = pltpu.async_copy(stab_hbm.at[nidx_v.at[j]], srow[s], sem_s[s])
                ca = pltpu.async_copy(atab_sh.at[aidx_v.at[j]], arow[s], sem_a[s])
                copies[s] = (cs, ca)
            if j >= 1:
                p = (j - 1) % 2
                cs, ca = copies[p]
                cs.wait()
                ca.wait()
                off = base + (j - 1) * CH
                pltpu.sync_copy(srow[p], semb_hbm.at[pl.ds(off, CH)])
                pltpu.sync_copy(arow[p], aemb_hbm.at[pl.ds(off, CH)])

    return _sc_gather


def kernel(node_ids, attrs, struct_table, attr_table, attr_fc_w, attr_fc_b, fusion_w, fusion_b):
    mesh = plsc.VectorSubcoreMesh(core_axis_name="c", subcore_axis_name="s",
                                  num_cores=NC, num_subcores=NS)
    node2d = node_ids.reshape(B // CH, CH)
    attr2d = attrs.reshape(B // CH, CH)

    def scratch(nch):
        return [
            pltpu.VMEM((nch, CH), jnp.int32),
            pltpu.VMEM((nch, CH), jnp.int32),
            pltpu.VMEM((CH, D), jnp.float32),
            pltpu.VMEM((CH, D), jnp.float32),
            pltpu.VMEM((CH, D), jnp.float32),
            pltpu.VMEM((CH, D), jnp.float32),
            pltpu.VMEM_SHARED((1001, D), jnp.float32),
            pltpu.SemaphoreType.DMA,
            pltpu.SemaphoreType.DMA,
            pltpu.SemaphoreType.DMA,
            pltpu.SemaphoreType.DMA,
        ]

    slices = []
    off = 0
    for bh in SLICES:
        nch = bh // NW // CH
        slices.append((off, bh,
                       pl.kernel(
                           _make_sc_gather(off // CH, nch),
                           out_type=[
                               jax.ShapeDtypeStruct((bh, D), jnp.float32),
                               jax.ShapeDtypeStruct((bh, D), jnp.float32),
                           ],
                           mesh=mesh,
                           scratch_types=scratch(nch),
                       )(node2d, attr2d, struct_table, attr_table)))
        off += bh

    ba2 = attr_fc_b.reshape(1, D)
    bf2 = fusion_b.reshape(1, D)
    out = None
    for k, (off, bh, (semb_h, aemb_h)) in enumerate(slices):
        BLK = min(4096, bh // 2)
        nblk = bh // BLK
        blk0 = off // BLK
        if k == 0:
            out = pl.pallas_call(
                _fuse_body,
                grid=(nblk,),
                in_specs=[
                    pl.BlockSpec((BLK, D), lambda i: (i, 0)),
                    pl.BlockSpec((BLK, D), lambda i: (i, 0)),
                    pl.BlockSpec((D, D), lambda i: (0, 0)),
                    pl.BlockSpec((1, D), lambda i: (0, 0)),
                    pl.BlockSpec((2 * D, D), lambda i: (0, 0)),
                    pl.BlockSpec((1, D), lambda i: (0, 0)),
                ],
                out_specs=pl.BlockSpec((BLK, D), lambda i: (i, 0)),
                out_shape=jax.ShapeDtypeStruct((B, D), jnp.float32),
            )(semb_h, aemb_h, attr_fc_w, ba2, fusion_w, bf2)
        else:
            out = pl.pallas_call(
                _fuse_body_aliased,
                grid=(nblk,),
                in_specs=[
                    pl.BlockSpec(memory_space=pltpu.MemorySpace.HBM),
                    pl.BlockSpec((BLK, D), lambda i: (i, 0)),
                    pl.BlockSpec((BLK, D), lambda i: (i, 0)),
                    pl.BlockSpec((D, D), lambda i: (0, 0)),
                    pl.BlockSpec((1, D), lambda i: (0, 0)),
                    pl.BlockSpec((2 * D, D), lambda i: (0, 0)),
                    pl.BlockSpec((1, D), lambda i: (0, 0)),
                ],
                out_specs=pl.BlockSpec((BLK, D),
                                       lambda i, blk0=blk0: (i + blk0, 0)),
                out_shape=jax.ShapeDtypeStruct((B, D), jnp.float32),
                input_output_aliases={0: 0},
            )(out, semb_h, aemb_h, attr_fc_w, ba2, fusion_w, bf2)
    return out


# speedup vs baseline: 1.0708x; 1.0058x over previous
import jax
import jax.numpy as jnp
from jax import lax
from jax.experimental import pallas as pl
from jax.experimental.pallas import tpu as pltpu
from jax.experimental.pallas import tpu_sc as plsc

B = 16384
D = 128
NC = 2
NS = 16
NW = NC * NS
CH = 128
SLICES = (8192, 8192)


def _fuse_math(semb, aemb, wa_ref, ba_ref, fw_ref, bf_ref):
    w1 = fw_ref[:D, :]
    w2 = fw_ref[D:, :]
    m = jnp.dot(wa_ref[...], w2, preferred_element_type=jnp.float32)
    c = jnp.dot(ba_ref[...], w2, preferred_element_type=jnp.float32) + bf_ref[...]
    return (
        jnp.dot(semb, w1, preferred_element_type=jnp.float32)
        + jnp.dot(aemb, m, preferred_element_type=jnp.float32)
        + c
    )


def _fuse_body(semb_ref, aemb_ref, wa_ref, ba_ref, fw_ref, bf_ref, out_ref):
    out_ref[...] = _fuse_math(semb_ref[...], aemb_ref[...], wa_ref, ba_ref,
                              fw_ref, bf_ref)


def _fuse_body_aliased(prev_ref, semb_ref, aemb_ref, wa_ref, ba_ref, fw_ref,
                       bf_ref, out_ref):
    del prev_ref
    out_ref[...] = _fuse_math(semb_ref[...], aemb_ref[...], wa_ref, ba_ref,
                              fw_ref, bf_ref)


def _make_sc_gather(row0, nch):
    bpw = nch * CH

    def _sc_gather(node_hbm, attr_hbm, stab_hbm, atab_hbm, semb_hbm, aemb_hbm,
                   nidx_v, aidx_v, srow0, srow1, arow0, arow1, atab_sh,
                   sem_s0, sem_s1, sem_a0, sem_a1, sem_t0, sem_t1):
        wid = lax.axis_index("s") * NC + lax.axis_index("c")
        sid = lax.axis_index("s")
        base = wid * bpw
        srow = (srow0, srow1)
        arow = (arow0, arow1)
        sem_s = (sem_s0, sem_s1)
        sem_a = (sem_a0, sem_a1)
        sem_t = (sem_t0, sem_t1)
        @pl.when(sid == 0)
        def _stage():
            pltpu.sync_copy(atab_hbm, atab_sh)

        r = row0 + wid * nch
        pltpu.sync_copy(node_hbm.at[pl.ds(r, nch)], nidx_v)
        pltpu.sync_copy(attr_hbm.at[pl.ds(r, nch)], aidx_v)
        plsc.subcore_barrier()
        copies = [None, None]
        stores = [None, None]
        for j in range(nch + 1):
            if j < nch:
                s = j % 2
                if stores[s] is not None:
                    stores[s][0].wait()
                    stores[s][1].wait()
                    stores[s] = None
                cs = pltpu.async_copy(stab_hbm.at[nidx_v.at[j]], srow[s], sem_s[s])
                ca = pltpu.async_copy(atab_sh.at[aidx_v.at[j]], arow[s], sem_a[s])
                copies[s] = (cs, ca)
            if j >= 1:
                p = (j - 1) % 2
                cs, ca = copies[p]
                cs.wait()
                ca.wait()
                off = base + (j - 1) * CH
                st1 = pltpu.async_copy(srow[p], semb_hbm.at[pl.ds(off, CH)], sem_t[p])
                st2 = pltpu.async_copy(arow[p], aemb_hbm.at[pl.ds(off, CH)], sem_t[p])
                stores[p] = (st1, st2)
        for sp in stores:
            if sp is not None:
                sp[0].wait()
                sp[1].wait()

    return _sc_gather


def kernel(node_ids, attrs, struct_table, attr_table, attr_fc_w, attr_fc_b, fusion_w, fusion_b):
    mesh = plsc.VectorSubcoreMesh(core_axis_name="c", subcore_axis_name="s",
                                  num_cores=NC, num_subcores=NS)
    node2d = node_ids.reshape(B // CH, CH)
    attr2d = attrs.reshape(B // CH, CH)

    def scratch(nch):
        return [
            pltpu.VMEM((nch, CH), jnp.int32),
            pltpu.VMEM((nch, CH), jnp.int32),
            pltpu.VMEM((CH, D), jnp.float32),
            pltpu.VMEM((CH, D), jnp.float32),
            pltpu.VMEM((CH, D), jnp.float32),
            pltpu.VMEM((CH, D), jnp.float32),
            pltpu.VMEM_SHARED((1001, D), jnp.float32),
            pltpu.SemaphoreType.DMA,
            pltpu.SemaphoreType.DMA,
            pltpu.SemaphoreType.DMA,
            pltpu.SemaphoreType.DMA,
            pltpu.SemaphoreType.DMA,
            pltpu.SemaphoreType.DMA,
        ]

    slices = []
    off = 0
    for bh in SLICES:
        nch = bh // NW // CH
        slices.append((off, bh,
                       pl.kernel(
                           _make_sc_gather(off // CH, nch),
                           out_type=[
                               jax.ShapeDtypeStruct((bh, D), jnp.float32),
                               jax.ShapeDtypeStruct((bh, D), jnp.float32),
                           ],
                           mesh=mesh,
                           scratch_types=scratch(nch),
                       )(node2d, attr2d, struct_table, attr_table)))
        off += bh

    ba2 = attr_fc_b.reshape(1, D)
    bf2 = fusion_b.reshape(1, D)
    out = None
    for k, (off, bh, (semb_h, aemb_h)) in enumerate(slices):
        BLK = min(4096, bh // 2)
        nblk = bh // BLK
        blk0 = off // BLK
        if k == 0:
            out = pl.pallas_call(
                _fuse_body,
                grid=(nblk,),
                in_specs=[
                    pl.BlockSpec((BLK, D), lambda i: (i, 0)),
                    pl.BlockSpec((BLK, D), lambda i: (i, 0)),
                    pl.BlockSpec((D, D), lambda i: (0, 0)),
                    pl.BlockSpec((1, D), lambda i: (0, 0)),
                    pl.BlockSpec((2 * D, D), lambda i: (0, 0)),
                    pl.BlockSpec((1, D), lambda i: (0, 0)),
                ],
                out_specs=pl.BlockSpec((BLK, D), lambda i: (i, 0)),
                out_shape=jax.ShapeDtypeStruct((B, D), jnp.float32),
            )(semb_h, aemb_h, attr_fc_w, ba2, fusion_w, bf2)
        else:
            out = pl.pallas_call(
                _fuse_body_aliased,
                grid=(nblk,),
                in_specs=[
                    pl.BlockSpec(memory_space=pltpu.MemorySpace.HBM),
                    pl.BlockSpec((BLK, D), lambda i: (i, 0)),
                    pl.BlockSpec((BLK, D), lambda i: (i, 0)),
                    pl.BlockSpec((D, D), lambda i: (0, 0)),
                    pl.BlockSpec((1, D), lambda i: (0, 0)),
                    pl.BlockSpec((2 * D, D), lambda i: (0, 0)),
                    pl.BlockSpec((1, D), lambda i: (0, 0)),
                ],
                out_specs=pl.BlockSpec((BLK, D),
                                       lambda i, blk0=blk0: (i + blk0, 0)),
                out_shape=jax.ShapeDtypeStruct((B, D), jnp.float32),
                input_output_aliases={0: 0},
            )(out, semb_h, aemb_h, attr_fc_w, ba2, fusion_w, bf2)
    return out
